# baseline (speedup 1.0000x reference)
import jax
import jax.numpy as jnp
from jax import lax
from jax.experimental import pallas as pl
from jax.experimental.pallas import tpu as pltpu
from jax.experimental.pallas import tpu_sc as plsc

D = 32
HIST = 200
N_ITEMS = 50
GH = 1000
EP_LEN = 200

ITEM_PAD = 64
HIST_PAD = 216
GH_PAD = 1008
ROWS_PER_TILE = 8
N_HTILES = HIST // ROWS_PER_TILE

OUT_LEN = N_ITEMS * D + HIST * D + D + EP_LEN
SACH_OFF = N_ITEMS * D
SUC_OFF = SACH_OFF + HIST * D
PRED_OFF = SUC_OFF + D

M_IROW = 0
M_IOFF = 64
M_HROW = 128
M_HOFF = 344
M_GH = 560
M_RAT = 1568
M_UROW = 1784
M_UOFF = 1792
M_CROW = 1800
M_COFF = 1808
M_LEN = 1816

NEG_BIG = -1e30


def _body(meta_h, noi_h, preds_h, utab_h, rtab_h, ctab_h, out_h,
          meta_v, noi_v, item128_v, hist128_v, user128_v, cat128_v,
          item_v, itemT_v, sach_v, sui_v, suc_v, pred_v,
          sem_a, sem_c):
    c = lax.axis_index("c")
    s = lax.axis_index("s")
    wid = s * 2 + c

    base = pl.multiple_of(jnp.minimum(wid * ROWS_PER_TILE, HIST), 8)
    sui_off = pl.multiple_of(jnp.clip((wid - N_HTILES) * 16, 0, 48), 8)

    pltpu.async_copy(meta_h, meta_v, sem_a).wait()

    g_item = pltpu.async_copy(rtab_h.at[meta_v.at[pl.ds(M_IROW, ITEM_PAD)]],
                              item128_v, sem_c)
    g_hist = pltpu.async_copy(
        rtab_h.at[meta_v.at[pl.ds(pl.multiple_of(M_HROW + base, 8), 8)]],
        hist128_v, sem_c)
    g_user = pltpu.async_copy(utab_h.at[meta_v.at[pl.ds(M_UROW, 8)]],
                              user128_v, sem_c)
    g_cat = pltpu.async_copy(ctab_h.at[meta_v.at[pl.ds(M_CROW, 8)]],
                             cat128_v, sem_c)
    cp_n = pltpu.async_copy(noi_h, noi_v, sem_a)

    s1 = jnp.zeros((16,), jnp.float32)
    s2 = jnp.zeros((16,), jnp.float32)
    for b in range(13):
        rf = meta_v[pl.ds(M_RAT + b * 16, 16)].astype(jnp.float32)
        s1 = s1 + rf
        s2 = s2 + rf * rf
    S1 = jnp.sum(s1)
    S2 = jnp.sum(s2)
    r_hist = jnp.float32(1.0 / HIST)
    rmean = S1 * r_hist
    rvar = (S2 - S1 * S1 * r_hist) * jnp.float32(1.0 / (HIST - 1))

    g_item.wait()
    g_hist.wait()
    g_user.wait()
    g_cat.wait()
    cp_n.wait()

    iota = lax.iota(jnp.int32, 16)
    d_lo = iota
    d_hi = iota + 16

    ioffs = [meta_v[pl.ds(M_IOFF + k * 16, 16)] for k in range(4)]
    for j in range(ITEM_PAD):
        jv = jnp.full((16,), j, jnp.int32)
        off = ioffs[j // 16][j % 16]
        vlo = plsc.load_gather(item128_v, [jv, off + iota])
        vhi = plsc.load_gather(item128_v, [jv, off + 16 + iota])
        item_v[j, pl.ds(0, 16)] = vlo
        item_v[j, pl.ds(16, 16)] = vhi
        plsc.store_scatter(itemT_v, [d_lo, jv], vlo)
        plsc.store_scatter(itemT_v, [d_hi, jv], vhi)

    lane_ok = [(iota + 16 * k) < N_ITEMS for k in range(4)]

    @pl.when(wid < N_HTILES)
    def _():
        ratw = meta_v[pl.ds(pl.multiple_of(M_RAT + base, 8), 16)].astype(
            jnp.float32)
        hidw = meta_v[pl.ds(pl.multiple_of(M_HROW + base, 8), 16)]
        hoffw = meta_v[pl.ds(pl.multiple_of(M_HOFF + base, 8), 16)]
        noiw = noi_v[pl.ds(base, 16)]
        for hh in range(ROWS_PER_TILE):
            mask_s = ((5.0 - ratw[hh]) * 0.2
                      + (rmean * 0.2 + rvar * noiw[hh]) * 0.2)
            hid = hidw[hh]
            hoff = hoffw[hh]
            acc = jnp.zeros((16,), jnp.float32)
            for b in range(GH_PAD // 16):
                acc = acc + jnp.where(
                    meta_v[pl.ds(M_GH + b * 16, 16)] == hid, 1.0, 0.0)
            cnt = jnp.sum(acc)
            m = mask_s * (1.0 - cnt * 0.1)

            hv = jnp.full((16,), hh, jnp.int32)
            hr_lo = plsc.load_gather(hist128_v, [hv, hoff + iota]) * m
            hr_hi = plsc.load_gather(hist128_v, [hv, hoff + 16 + iota]) * m
            l0 = jnp.zeros((16,), jnp.float32)
            l1 = jnp.zeros((16,), jnp.float32)
            l2 = jnp.zeros((16,), jnp.float32)
            l3 = jnp.zeros((16,), jnp.float32)
            for d in range(D):
                sc = hr_lo[d] if d < 16 else hr_hi[d - 16]
                l0 = l0 + sc * itemT_v[d, pl.ds(0, 16)]
                l1 = l1 + sc * itemT_v[d, pl.ds(16, 16)]
                l2 = l2 + sc * itemT_v[d, pl.ds(32, 16)]
                l3 = l3 + sc * itemT_v[d, pl.ds(48, 16)]
            l0 = jnp.where(lane_ok[0], l0, NEG_BIG)
            l1 = jnp.where(lane_ok[1], l1, NEG_BIG)
            l2 = jnp.where(lane_ok[2], l2, NEG_BIG)
            l3 = jnp.where(lane_ok[3], l3, NEG_BIG)
            mx = jnp.max(jnp.maximum(jnp.maximum(l0, l1),
                                     jnp.maximum(l2, l3)))
            es = [jnp.exp(l0 - mx), jnp.exp(l1 - mx),
                  jnp.exp(l2 - mx), jnp.exp(l3 - mx)]
            z = jnp.sum(es[0] + es[1] + es[2] + es[3])
            a_lo = jnp.zeros((16,), jnp.float32)
            a_hi = jnp.zeros((16,), jnp.float32)
            for j in range(N_ITEMS):
                aj = es[j // 16][j % 16]
                a_lo = a_lo + aj * item_v[j, pl.ds(0, 16)]
                a_hi = a_hi + aj * item_v[j, pl.ds(16, 16)]
            sach_v[pl.ds(hh * D, 16)] = a_lo / z
            sach_v[pl.ds(hh * D + 16, 16)] = a_hi / z
        pltpu.sync_copy(
            sach_v,
            out_h.at[pl.ds(SACH_OFF + wid * (ROWS_PER_TILE * D),
                           ROWS_PER_TILE * D)])

    @pl.when(jnp.logical_and(wid >= N_HTILES, wid <= 28))
    def _():
        wu = meta_v[pl.ds(M_UROW, 16)]
        uoff = wu[8]
        zv = jnp.zeros((16,), jnp.int32)
        u_lo = plsc.load_gather(user128_v, [zv, uoff + iota])
        u_hi = plsc.load_gather(user128_v, [zv, uoff + 16 + iota])
        for jj in range(16):
            jv = jnp.full((16,), sui_off + jj, jnp.int32)
            v_lo = plsc.load_gather(item_v, [jv, iota])
            v_hi = plsc.load_gather(item_v, [jv, iota + 16])
            sui_v[pl.ds(jj * D, 16)] = u_lo * v_lo
            sui_v[pl.ds(jj * D + 16, 16)] = u_hi * v_hi

    @pl.when(jnp.logical_and(wid >= N_HTILES, wid <= 27))
    def _():
        pltpu.sync_copy(sui_v, out_h.at[pl.ds((wid - N_HTILES) * (16 * D),
                                              16 * D)])

    @pl.when(wid == 28)
    def _():
        pltpu.sync_copy(sui_v.at[pl.ds(0, 2 * D)],
                        out_h.at[pl.ds(48 * D, 2 * D)])

    @pl.when(wid == 30)
    def _():
        pltpu.sync_copy(preds_h, pred_v)
        pltpu.sync_copy(pred_v, out_h.at[pl.ds(PRED_OFF, EP_LEN)])

    @pl.when(wid == 31)
    def _():
        wu = meta_v[pl.ds(M_UROW, 16)]
        wc = meta_v[pl.ds(M_CROW, 16)]
        uoff = wu[8]
        coff = wc[8]
        zv = jnp.zeros((16,), jnp.int32)
        u_lo = plsc.load_gather(user128_v, [zv, uoff + iota])
        u_hi = plsc.load_gather(user128_v, [zv, uoff + 16 + iota])
        c_lo = plsc.load_gather(cat128_v, [zv, coff + iota])
        c_hi = plsc.load_gather(cat128_v, [zv, coff + 16 + iota])
        suc_v[pl.ds(0, 16)] = u_lo * c_lo
        suc_v[pl.ds(16, 16)] = u_hi * c_hi
        pltpu.sync_copy(suc_v, out_h.at[pl.ds(SUC_OFF, D)])


@jax.jit
def _sc_forward(meta, noi, preds, ut128, rt128, ct128):
    mesh = plsc.VectorSubcoreMesh(core_axis_name="c", subcore_axis_name="s")
    f = pl.kernel(
        _body,
        out_type=jax.ShapeDtypeStruct((OUT_LEN,), jnp.float32),
        mesh=mesh,
        compiler_params=pltpu.CompilerParams(needs_layout_passes=False,
                                             use_tc_tiling_on_sc=False),
        scratch_types=[
            pltpu.VMEM((M_LEN,), jnp.int32),
            pltpu.VMEM((HIST_PAD,), jnp.float32),
            pltpu.VMEM((ITEM_PAD, 128), jnp.float32),
            pltpu.VMEM((8, 128), jnp.float32),
            pltpu.VMEM((8, 128), jnp.float32),
            pltpu.VMEM((8, 128), jnp.float32),
            pltpu.VMEM((ITEM_PAD, D), jnp.float32),
            pltpu.VMEM((D, ITEM_PAD), jnp.float32),
            pltpu.VMEM((ROWS_PER_TILE * D,), jnp.float32),
            pltpu.VMEM((16 * D,), jnp.float32),
            pltpu.VMEM((D,), jnp.float32),
            pltpu.VMEM((EP_LEN,), jnp.float32),
            pltpu.SemaphoreType.DMA,
            pltpu.SemaphoreType.DMA,
        ],
    )
    return f(meta, noi, preds, ut128, rt128, ct128)


def kernel(user_ids, item_id, idx, history, global_history, rating, preds,
           last_category, repetition, user_table, recipe_table,
           category_table):
    i32 = jnp.int32
    uid = jnp.asarray(user_ids, i32)
    lc = jnp.asarray(last_category, i32) - 1
    iidx = item_id.astype(i32)
    hidx = history.astype(i32)
    zpad_i = jnp.zeros((ITEM_PAD - N_ITEMS,), i32)
    zpad_h = jnp.zeros((HIST_PAD - HIST,), i32)
    meta = jnp.concatenate([
        jnp.concatenate([iidx >> 2, zpad_i]),
        jnp.concatenate([(iidx & 3) * D, zpad_i]),
        jnp.concatenate([hidx >> 2, zpad_h]),
        jnp.concatenate([(hidx & 3) * D, zpad_h]),
        jnp.concatenate([global_history.astype(i32),
                         jnp.full((GH_PAD - GH,), -1, i32)]),
        jnp.concatenate([rating.astype(i32), zpad_h]),
        jnp.full((8,), uid >> 2, i32),
        jnp.full((8,), (uid & 3) * D, i32),
        jnp.full((8,), lc >> 2, i32),
        jnp.full((8,), (lc & 3) * D, i32),
    ])
    noise = jax.random.normal(jax.random.key(42), (HIST,), dtype=jnp.float32)
    noi = jnp.concatenate([noise, jnp.zeros((HIST_PAD - HIST,), jnp.float32)])
    out = _sc_forward(meta, noi, preds.astype(jnp.float32),
                      user_table.reshape(-1, 128),
                      recipe_table.reshape(-1, 128),
                      jnp.concatenate(
                          [category_table,
                           jnp.zeros((14, D), jnp.float32)]).reshape(-1, 128))
    return out.reshape(1, OUT_LEN)


# speedup vs baseline: 13.2018x; 13.2018x over previous
import jax
import jax.numpy as jnp
from jax import lax
from jax.experimental import pallas as pl
from jax.experimental.pallas import tpu as pltpu
from jax.experimental.pallas import tpu_sc as plsc

D = 32
HIST = 200
N_ITEMS = 50
GH = 1000
EP_LEN = 200
N_RECIPES = 1000000
N_USERS = 100000

ITEM_PAD = 64
HIST_PAD = 216
GH_PAD = 1008
ROWS_PER_TILE = 8
N_HTILES = HIST // ROWS_PER_TILE

OUT_LEN = N_ITEMS * D + HIST * D + D + EP_LEN
SACH_OFF = N_ITEMS * D
SUC_OFF = SACH_OFF + HIST * D
PRED_OFF = SUC_OFF + D

M_ICB = 0
M_IC = 64
M_HCB = 128
M_HC = 344
M_GH = 560
M_RAT = 1568
M_UCB = 1784
M_UC = 1792
M_CC = 1800
M_LEN = 1824

NEG_BIG = -1e30


def _body(meta_h, noi_h, preds_h, utabT_h, rtabT_h, catT_h, out_h,
          meta_v, noi_v, istrip_v, hstrip_v, ustrip_v, cstrip_v,
          item_v, itemT_v, colstage_v, sach_v, sui_v, suc_v, pred_v,
          item_sh,
          sem_a, sem_h, sem_i, sem_u):
    cid = lax.axis_index("c")
    sid = lax.axis_index("s")
    wid = sid * 2 + cid

    base = pl.multiple_of(jnp.minimum(wid * ROWS_PER_TILE, HIST), 8)
    sui_off = pl.multiple_of(jnp.clip((wid - N_HTILES) * 16, 0, 48), 8)

    pltpu.async_copy(meta_h, meta_v, sem_a).wait()
    cp_n = pltpu.async_copy(noi_h, noi_v, sem_a)

    iota = lax.iota(jnp.int32, 16)

    hcbw = meta_v[pl.ds(pl.multiple_of(M_HCB + base, 8), 16)]
    h_cps = []
    for hh in range(ROWS_PER_TILE):
        cb = pl.multiple_of(hcbw[hh], 128)
        h_cps.append(pltpu.async_copy(rtabT_h.at[:, pl.ds(cb, 128)],
                                      hstrip_v.at[hh], sem_h))

    @pl.when(sid < 8)
    def _():
        icbw = meta_v[pl.ds(pl.multiple_of(M_ICB + 8 * sid, 8), 16)]
        icw = meta_v[pl.ds(pl.multiple_of(M_IC + 8 * sid, 8), 16)]
        cps = []
        for k in range(8):
            cb = pl.multiple_of(icbw[k], 128)
            cps.append(pltpu.async_copy(rtabT_h.at[:, pl.ds(cb, 128)],
                                        istrip_v.at[k], sem_i))
        for cp in cps:
            cp.wait()
        for k in range(8):
            kv = jnp.full((16,), k, jnp.int32)
            cv = jnp.full((16,), icw[k], jnp.int32)
            colstage_v[pl.ds(0, 16)] = plsc.load_gather(
                istrip_v, [kv, iota, cv])
            colstage_v[pl.ds(16, 16)] = plsc.load_gather(
                istrip_v, [kv, iota + 16, cv])
            pltpu.sync_copy(colstage_v, item_sh.at[8 * sid + k])

    plsc.subcore_barrier()
    pltpu.sync_copy(item_sh, item_v)

    uw = meta_v[pl.ds(M_UCB, 16)]
    g_user = pltpu.async_copy(
        utabT_h.at[:, pl.ds(pl.multiple_of(uw[0], 128), 128)], ustrip_v, sem_u)
    g_cat = pltpu.async_copy(catT_h, cstrip_v, sem_u)

    s1 = jnp.zeros((16,), jnp.float32)
    s2 = jnp.zeros((16,), jnp.float32)
    for b in range(13):
        rf = meta_v[pl.ds(M_RAT + b * 16, 16)].astype(jnp.float32)
        s1 = s1 + rf
        s2 = s2 + rf * rf
    S1 = jnp.sum(s1)
    S2 = jnp.sum(s2)
    r_hist = jnp.float32(1.0 / HIST)
    rmean = S1 * r_hist
    rvar = (S2 - S1 * S1 * r_hist) * jnp.float32(1.0 / (HIST - 1))

    d_lo = iota
    d_hi = iota + 16
    for j in range(ITEM_PAD):
        jv = jnp.full((16,), j, jnp.int32)
        plsc.store_scatter(itemT_v, [d_lo, jv], item_v[j, pl.ds(0, 16)])
        plsc.store_scatter(itemT_v, [d_hi, jv], item_v[j, pl.ds(16, 16)])

    lane_ok = [(iota + 16 * k) < N_ITEMS for k in range(4)]

    for cp in h_cps:
        cp.wait()
    cp_n.wait()

    @pl.when(wid < N_HTILES)
    def _():
        ratw = meta_v[pl.ds(pl.multiple_of(M_RAT + base, 8), 16)].astype(
            jnp.float32)
        hcw = meta_v[pl.ds(pl.multiple_of(M_HC + base, 8), 16)]
        noiw = noi_v[pl.ds(base, 16)]
        hcbw2 = meta_v[pl.ds(pl.multiple_of(M_HCB + base, 8), 16)]
        for hh in range(ROWS_PER_TILE):
            mask_s = ((5.0 - ratw[hh]) * 0.2
                      + (rmean * 0.2 + rvar * noiw[hh]) * 0.2)
            hid = hcbw2[hh] + hcw[hh]
            acc = jnp.zeros((16,), jnp.float32)
            for b in range(GH_PAD // 16):
                acc = acc + jnp.where(
                    meta_v[pl.ds(M_GH + b * 16, 16)] == hid, 1.0, 0.0)
            cnt = jnp.sum(acc)
            m = mask_s * (1.0 - cnt * 0.1)

            hv = jnp.full((16,), hh, jnp.int32)
            cv = jnp.full((16,), hcw[hh], jnp.int32)
            hr_lo = plsc.load_gather(hstrip_v, [hv, iota, cv]) * m
            hr_hi = plsc.load_gather(hstrip_v, [hv, iota + 16, cv]) * m
            l0 = jnp.zeros((16,), jnp.float32)
            l1 = jnp.zeros((16,), jnp.float32)
            l2 = jnp.zeros((16,), jnp.float32)
            l3 = jnp.zeros((16,), jnp.float32)
            for d in range(D):
                sc = hr_lo[d] if d < 16 else hr_hi[d - 16]
                l0 = l0 + sc * itemT_v[d, pl.ds(0, 16)]
                l1 = l1 + sc * itemT_v[d, pl.ds(16, 16)]
                l2 = l2 + sc * itemT_v[d, pl.ds(32, 16)]
                l3 = l3 + sc * itemT_v[d, pl.ds(48, 16)]
            l0 = jnp.where(lane_ok[0], l0, NEG_BIG)
            l1 = jnp.where(lane_ok[1], l1, NEG_BIG)
            l2 = jnp.where(lane_ok[2], l2, NEG_BIG)
            l3 = jnp.where(lane_ok[3], l3, NEG_BIG)
            mx = jnp.max(jnp.maximum(jnp.maximum(l0, l1),
                                     jnp.maximum(l2, l3)))
            es = [jnp.exp(l0 - mx), jnp.exp(l1 - mx),
                  jnp.exp(l2 - mx), jnp.exp(l3 - mx)]
            z = jnp.sum(es[0] + es[1] + es[2] + es[3])
            a_lo = jnp.zeros((16,), jnp.float32)
            a_hi = jnp.zeros((16,), jnp.float32)
            for j in range(N_ITEMS):
                aj = es[j // 16][j % 16]
                a_lo = a_lo + aj * item_v[j, pl.ds(0, 16)]
                a_hi = a_hi + aj * item_v[j, pl.ds(16, 16)]
            sach_v[pl.ds(hh * D, 16)] = a_lo / z
            sach_v[pl.ds(hh * D + 16, 16)] = a_hi / z
        pltpu.sync_copy(
            sach_v,
            out_h.at[pl.ds(SACH_OFF + wid * (ROWS_PER_TILE * D),
                           ROWS_PER_TILE * D)])

    g_user.wait()
    g_cat.wait()

    @pl.when(jnp.logical_and(wid >= N_HTILES, wid <= 28))
    def _():
        ucv = jnp.full((16,), uw[8], jnp.int32)
        u_lo = plsc.load_gather(ustrip_v, [iota, ucv])
        u_hi = plsc.load_gather(ustrip_v, [iota + 16, ucv])
        for jj in range(16):
            jv = jnp.full((16,), sui_off + jj, jnp.int32)
            v_lo = plsc.load_gather(item_v, [jv, iota])
            v_hi = plsc.load_gather(item_v, [jv, iota + 16])
            sui_v[pl.ds(jj * D, 16)] = u_lo * v_lo
            sui_v[pl.ds(jj * D + 16, 16)] = u_hi * v_hi

    @pl.when(jnp.logical_and(wid >= N_HTILES, wid <= 27))
    def _():
        pltpu.sync_copy(sui_v, out_h.at[pl.ds((wid - N_HTILES) * (16 * D),
                                              16 * D)])

    @pl.when(wid == 28)
    def _():
        pltpu.sync_copy(sui_v.at[pl.ds(0, 2 * D)],
                        out_h.at[pl.ds(48 * D, 2 * D)])

    @pl.when(wid == 30)
    def _():
        pltpu.sync_copy(preds_h, pred_v)
        pltpu.sync_copy(pred_v, out_h.at[pl.ds(PRED_OFF, EP_LEN)])

    @pl.when(wid == 31)
    def _():
        wc = meta_v[pl.ds(M_CC, 16)]
        ucv = jnp.full((16,), uw[8], jnp.int32)
        ccv = jnp.full((16,), wc[0], jnp.int32)
        u_lo = plsc.load_gather(ustrip_v, [iota, ucv])
        u_hi = plsc.load_gather(ustrip_v, [iota + 16, ucv])
        c_lo = plsc.load_gather(cstrip_v, [iota, ccv])
        c_hi = plsc.load_gather(cstrip_v, [iota + 16, ccv])
        suc_v[pl.ds(0, 16)] = u_lo * c_lo
        suc_v[pl.ds(16, 16)] = u_hi * c_hi
        pltpu.sync_copy(suc_v, out_h.at[pl.ds(SUC_OFF, D)])


@jax.jit
def _sc_forward(meta, noi, preds, utabT, rtabT, catT):
    mesh = plsc.VectorSubcoreMesh(core_axis_name="c", subcore_axis_name="s")
    f = pl.kernel(
        _body,
        out_type=jax.ShapeDtypeStruct((OUT_LEN,), jnp.float32),
        mesh=mesh,
        compiler_params=pltpu.CompilerParams(needs_layout_passes=False,
                                             use_tc_tiling_on_sc=True),
        scratch_types=[
            pltpu.VMEM((M_LEN,), jnp.int32),
            pltpu.VMEM((HIST_PAD,), jnp.float32),
            pltpu.VMEM((8, D, 128), jnp.float32),
            pltpu.VMEM((ROWS_PER_TILE, D, 128), jnp.float32),
            pltpu.VMEM((D, 128), jnp.float32),
            pltpu.VMEM((D, 128), jnp.float32),
            pltpu.VMEM((ITEM_PAD, 128), jnp.float32),
            pltpu.VMEM((D, 128), jnp.float32),
            pltpu.VMEM((128,), jnp.float32),
            pltpu.VMEM((ROWS_PER_TILE * D,), jnp.float32),
            pltpu.VMEM((16 * D,), jnp.float32),
            pltpu.VMEM((D,), jnp.float32),
            pltpu.VMEM((EP_LEN,), jnp.float32),
            pltpu.VMEM_SHARED((ITEM_PAD, 128), jnp.float32),
            pltpu.SemaphoreType.DMA,
            pltpu.SemaphoreType.DMA,
            pltpu.SemaphoreType.DMA,
            pltpu.SemaphoreType.DMA,
        ],
    )
    return f(meta, noi, preds, utabT, rtabT, catT)


def _strip(ids):
    return (ids >> 7) << 7, ids & 127


def kernel(user_ids, item_id, idx, history, global_history, rating, preds,
           last_category, repetition, user_table, recipe_table,
           category_table):
    i32 = jnp.int32
    uid = jnp.asarray(user_ids, i32)
    lc = jnp.asarray(last_category, i32) - 1
    iidx = item_id.astype(i32)
    hidx = history.astype(i32)
    icb, ic = _strip(iidx)
    hcb, hc = _strip(hidx)
    ucb, uc = _strip(uid)
    zpad_i = jnp.zeros((ITEM_PAD - N_ITEMS,), i32)
    zpad_h = jnp.zeros((HIST_PAD - HIST,), i32)
    meta = jnp.concatenate([
        jnp.concatenate([icb, zpad_i]),
        jnp.concatenate([ic, zpad_i]),
        jnp.concatenate([hcb, zpad_h]),
        jnp.concatenate([hc, zpad_h]),
        jnp.concatenate([global_history.astype(i32),
                         jnp.full((GH_PAD - GH,), -1, i32)]),
        jnp.concatenate([rating.astype(i32), zpad_h]),
        jnp.full((8,), ucb, i32),
        jnp.full((8,), uc, i32),
        jnp.full((8,), lc, i32),
        jnp.zeros((16,), i32),
    ])
    noise = jax.random.normal(jax.random.key(42), (HIST,), dtype=jnp.float32)
    noi = jnp.concatenate([noise, jnp.zeros((HIST_PAD - HIST,), jnp.float32)])
    catT = jnp.concatenate(
        [category_table, jnp.zeros((128 - 50, D), jnp.float32)]).T
    out = _sc_forward(meta, noi, preds.astype(jnp.float32),
                      user_table.T, recipe_table.T, catT)
    return out.reshape(1, OUT_LEN)
